# Initial kernel scaffold; baseline (speedup 1.0000x reference)
import jax
import jax.numpy as jnp
from jax.experimental import pallas as pl


def kernel(x, table):
    raise NotImplementedError("write your pallas kernel here")


# speedup vs baseline: 1.4313x; 1.4313x over previous
import functools

import jax
import jax.numpy as jnp
import numpy as np
from jax import lax
from jax.experimental import pallas as pl
from jax.experimental.pallas import tpu as pltpu
from jax.experimental.pallas import tpu_sc as plsc

VOCAB = 100000
D_MODEL = 768
SEQ_LEN = 2048
BATCH = 4

NC = 2
NS = 16
NW = NC * NS

TOKENS = BATCH * SEQ_LEN
TOK_PER_W = TOKENS // NW
CHUNK = 64
NCHUNK = TOK_PER_W // CHUNK
LANES = 16
KSTEPS = D_MODEL // LANES
W_PER_SEQ = SEQ_LEN // TOK_PER_W


def _pos_encoding_np(seq_len, d_model):
    pos = np.arange(seq_len, dtype=np.float32)[:, None]
    ind = np.arange(0, d_model, 2, dtype=np.float32)
    angle = pos / (10000.0 ** (ind / d_model))
    enc = np.zeros((seq_len, d_model), dtype=np.float32)
    enc[:, 0::2] = np.sin(angle)
    enc[:, 1::2] = np.cos(angle)
    return enc


_POS_ENC = _pos_encoding_np(SEQ_LEN, D_MODEL)


def _sc_body(x_hbm, pos_hbm, table_hbm, out_hbm, idx_v, pos_v, rows_v, gsem):
    wid = lax.axis_index("s") * NC + lax.axis_index("c")
    base = wid * TOK_PER_W
    pos0 = (wid % W_PER_SEQ) * TOK_PER_W

    pltpu.sync_copy(x_hbm.at[wid], idx_v)

    for c in range(NCHUNK):
        cp = pltpu.async_copy(table_hbm.at[idx_v.at[c]], rows_v, gsem)
        pltpu.sync_copy(pos_hbm.at[pl.ds(pos0 + c * CHUNK, CHUNK)], pos_v)
        cp.wait()

        def add_row(j):
            for k in range(KSTEPS):
                sl = pl.ds(k * LANES, LANES)
                rows_v[j, sl] = rows_v[j, sl] + pos_v[j, sl]

        lax.fori_loop(0, CHUNK, lambda j, _: (add_row(j), 0)[1], 0)

        pltpu.sync_copy(rows_v, out_hbm.at[pl.ds(base + c * CHUNK, CHUNK)])


@jax.jit
def _embed(x_grouped, pos, table):
    mesh = plsc.VectorSubcoreMesh(
        core_axis_name="c", subcore_axis_name="s", num_cores=NC, num_subcores=NS
    )
    k = pl.kernel(
        _sc_body,
        out_type=jax.ShapeDtypeStruct((TOKENS, D_MODEL), jnp.float32),
        mesh=mesh,
        scratch_types=[
            pltpu.VMEM((NCHUNK, CHUNK), jnp.int32),
            pltpu.VMEM((CHUNK, D_MODEL), jnp.float32),
            pltpu.VMEM((CHUNK, D_MODEL), jnp.float32),
            pltpu.SemaphoreType.DMA,
        ],
    )
    return k(x_grouped, pos, table)


def kernel(x, table):
    x_grouped = x.reshape(NW, NCHUNK, CHUNK).astype(jnp.int32)
    pos = jnp.asarray(_POS_ENC)
    out = _embed(x_grouped, pos, table)
    return out.reshape(BATCH, SEQ_LEN, D_MODEL)
